# baseline (speedup 1.0000x reference)
import jax
import jax.numpy as jnp
from jax.experimental import pallas as pl

N = 100000
E = 3200000
DIMS = [(1, 32, 'add'), (32, 64, 'max'), (64, 32, 'mean'), (32, 64, 'max'), (64, 32, 'mean'), (32, 64, 'max'), (64, 32, 'mean'), (32, 1, 'max')]


def _agg(msgs, dst, num_nodes, aggr):
    if aggr == 'add':
        return jax.ops.segment_sum(msgs, dst, num_segments=num_nodes)
    if aggr == 'mean':
        s = jax.ops.segment_sum(msgs, dst, num_segments=num_nodes)
        cnt = jax.ops.segment_sum(jnp.ones((msgs.shape[0],), msgs.dtype), dst, num_segments=num_nodes)
        return s / jnp.clip(cnt, 1.0, None)[:, None]
    m = jax.ops.segment_max(msgs, dst, num_segments=num_nodes)
    return jnp.where(jnp.isfinite(m), m, 0.0)


def _sigmoid_kernel(x_ref, o_ref):
    o_ref[...] = jax.nn.sigmoid(x_ref[...])


def kernel(x, edge_index, params):
    src, dst = edge_index[0], edge_index[1]
    h = x.reshape(-1, 1)
    for i, (din, dout, aggr) in enumerate(DIMS):
        agg = _agg(h[src], dst, h.shape[0], aggr)
        h = agg @ params['Wr%d' % i].T + params['br%d' % i] + h @ params['Wl%d' % i].T
        if i < len(DIMS) - 1:
            h = jax.nn.relu(h)
    return pl.pallas_call(
        _sigmoid_kernel,
        grid=(10,),
        in_specs=[pl.BlockSpec((N // 10, 1), lambda i: (i, 0))],
        out_specs=pl.BlockSpec((N // 10, 1), lambda i: (i, 0)),
        out_shape=jax.ShapeDtypeStruct(h.shape, h.dtype),
    )(h)


# speedup vs baseline: 7.8712x; 7.8712x over previous
import functools

import jax
import jax.numpy as jnp
from jax import lax
from jax.experimental import pallas as pl
from jax.experimental.pallas import tpu as pltpu
from jax.experimental.pallas import tpu_sc as plsc

N = 100000
E = 3200000
NC = 2
NS = 16
NB = NC * NS
RPB = 3128
PADN = NB * RPB
HALFP = NS * RPB
ACCR = 50176
DUMMY = HALFP + 8
MAXR = 3136
BB = 3200
FLUSH = 2048
BUFCAP = 8192
CAP = E + FLUSH
PB = 128
TCB = 1000


def _s0(v):
    return lax.squeeze(lax.slice(v, (0,), (1,)), (0,))


_MESH = plsc.VectorSubcoreMesh(core_axis_name="c", subcore_axis_name="s")


@functools.partial(
    pl.kernel,
    out_type=(
        jax.ShapeDtypeStruct((NB * CAP,), jnp.int32),
        jax.ShapeDtypeStruct((NB * CAP,), jnp.int32),
        jax.ShapeDtypeStruct((NB * 16,), jnp.int32),
    ),
    mesh=_MESH,
    scratch_types=[
        pltpu.VMEM((BB,), jnp.int32),
        pltpu.VMEM((BB,), jnp.int32),
        pltpu.VMEM((BUFCAP,), jnp.int32),
        pltpu.VMEM((BUFCAP,), jnp.int32),
        pltpu.VMEM((16,), jnp.int32),
    ],
    compiler_params=pltpu.CompilerParams(needs_layout_passes=False),
)
def _bucketize(ei, bsrc, bdst, bcnt, esv, edv, bufs, bufd, cv):
    c = lax.axis_index("c")
    s = lax.axis_index("s")
    b = c * NS + s
    lo = b * RPB
    hi = lo + RPB
    halfbase = c * HALFP
    region = b * CAP
    iota = lax.iota(jnp.int32, 16)

    def scan_block(blk, carry):
        off, woff = carry
        base = pl.multiple_of(blk * BB, 128)
        pltpu.sync_copy(ei.at[pl.ds(base, BB)], esv)
        pltpu.sync_copy(ei.at[pl.ds(E + base, BB)], edv)

        def grp(g, off_spl):
            o16 = pl.multiple_of(g * 16, 16)
            sv = esv[pl.ds(o16, 16)]
            dv = edv[pl.ds(o16, 16)]
            m = (dv >= lo) & (dv < hi)
            mi = m.astype(jnp.int32)
            pos = off_spl + plsc.cumsum(mi) - mi
            plsc.store_scatter(bufs, [pos], sv, mask=m)
            plsc.store_scatter(bufd, [pos], dv - halfbase, mask=m)
            return off_spl + plsc.all_reduce_population_count(m)

        off_spl = lax.fori_loop(0, BB // 16, grp,
                                jnp.full((16,), off, jnp.int32))
        off = _s0(off_spl)

        def flush_body(cw):
            o, w = cw
            dst_off = pl.multiple_of(region + w, 128)
            pltpu.sync_copy(bufs.at[pl.ds(0, FLUSH)], bsrc.at[pl.ds(dst_off, FLUSH)])
            pltpu.sync_copy(bufd.at[pl.ds(0, FLUSH)], bdst.at[pl.ds(dst_off, FLUSH)])

            def shift(i, _):
                o16 = pl.multiple_of(i * 16, 16)
                bufs[pl.ds(o16, 16)] = bufs[pl.ds(FLUSH + o16, 16)]
                bufd[pl.ds(o16, 16)] = bufd[pl.ds(FLUSH + o16, 16)]
                return 0

            lax.fori_loop(0, (BUFCAP - FLUSH) // 16, shift, 0)
            return (o - FLUSH, w + FLUSH)

        off, woff = lax.while_loop(lambda cw: cw[0] >= FLUSH, flush_body,
                                   (off, woff))
        return (off, woff)

    off, woff = lax.fori_loop(0, E // BB, scan_block,
                              (jnp.int32(0), jnp.int32(0)))
    total = woff + off

    off_spl = jnp.full((16,), off, jnp.int32)
    zeros16 = jnp.zeros((16,), jnp.int32)
    dummy16 = jnp.full((16,), DUMMY, jnp.int32)

    def pad(g, _):
        pos = off_spl + g * 16 + iota
        plsc.store_scatter(bufs, [pos], zeros16)
        plsc.store_scatter(bufd, [pos], dummy16)
        return 0

    lax.fori_loop(0, FLUSH // 16, pad, 0)
    dst_off = pl.multiple_of(region + woff, 128)
    pltpu.sync_copy(bufs.at[pl.ds(0, FLUSH)], bsrc.at[pl.ds(dst_off, FLUSH)])
    pltpu.sync_copy(bufd.at[pl.ds(0, FLUSH)], bdst.at[pl.ds(dst_off, FLUSH)])
    cv[...] = jnp.full((16,), total, jnp.int32)
    pltpu.sync_copy(cv, bcnt.at[pl.ds(pl.multiple_of(b * 16, 16), 16)])


def _make_sum(d):
    @functools.partial(
        pl.kernel,
        out_type=jax.ShapeDtypeStruct((PADN, d), jnp.float32),
        mesh=_MESH,
        scratch_types=[
            pltpu.VMEM_SHARED((ACCR, d), jnp.float32),
            pltpu.VMEM((PB,), jnp.int32),
            pltpu.VMEM((PB,), jnp.int32),
            pltpu.VMEM((PB, d), jnp.float32),
            pltpu.VMEM((16,), jnp.int32),
            pltpu.SemaphoreType.DMA,
        ],
        compiler_params=pltpu.CompilerParams(needs_layout_passes=False,
                                             use_tc_tiling_on_sc=False),
    )
    def k(y, zeros, bsrc, bdst, bcnt, out, acc, idxv, dlocv, rows, cv, sem):
        c = lax.axis_index("c")
        s = lax.axis_index("s")
        b = c * NS + s
        region = b * CAP
        rpt = ACCR // NS
        zo = pl.multiple_of(s * rpt, 8)
        pltpu.sync_copy(zeros.at[pl.ds(zo, rpt)], acc.at[pl.ds(zo, rpt)])
        plsc.subcore_barrier()
        pltpu.sync_copy(bcnt.at[pl.ds(pl.multiple_of(b * 16, 16), 16)], cv)
        cnt = _s0(cv[...])
        nblk = (cnt + PB - 1) // PB

        def blk(kk, _):
            o = pl.multiple_of(region + kk * PB, 128)
            pltpu.sync_copy(bsrc.at[pl.ds(o, PB)], idxv)
            pltpu.sync_copy(bdst.at[pl.ds(o, PB)], dlocv)
            pltpu.async_copy(y.at[idxv], rows, sem).wait()
            pltpu.sync_copy(rows, acc.at[dlocv], add=True)
            return 0

        lax.fori_loop(0, nblk, blk, 0)
        plsc.subcore_barrier()
        oo = pl.multiple_of(s * RPB, 8)
        pltpu.sync_copy(acc.at[pl.ds(oo, RPB)],
                        out.at[pl.ds(pl.multiple_of(c * HALFP + oo, 8), RPB)])

    return k


_sum16 = _make_sum(16)
_sum32 = _make_sum(32)


@functools.partial(
    pl.kernel,
    out_type=jax.ShapeDtypeStruct((PADN, 32), jnp.float32),
    mesh=_MESH,
    scratch_types=[
        pltpu.VMEM((MAXR, 32), jnp.float32),
        pltpu.VMEM((PB,), jnp.int32),
        pltpu.VMEM((PB,), jnp.int32),
        pltpu.VMEM((PB, 32), jnp.float32),
        pltpu.VMEM((16,), jnp.int32),
        pltpu.SemaphoreType.DMA,
    ],
    compiler_params=pltpu.CompilerParams(needs_layout_passes=False,
                                         use_tc_tiling_on_sc=False),
)
def _max32(h, zeros, bsrc, bdst, bcnt, out, acc, idxv, lidx, rows, cv, sem):
    c = lax.axis_index("c")
    s = lax.axis_index("s")
    b = c * NS + s
    region = b * CAP
    srel = s * RPB
    pltpu.sync_copy(zeros.at[pl.ds(0, MAXR)], acc)
    pltpu.sync_copy(bcnt.at[pl.ds(pl.multiple_of(b * 16, 16), 16)], cv)
    cnt = _s0(cv[...])
    nblk = (cnt + PB - 1) // PB

    def blk(kk, _):
        o = pl.multiple_of(region + kk * PB, 128)
        pltpu.sync_copy(bsrc.at[pl.ds(o, PB)], idxv)
        pltpu.sync_copy(bdst.at[pl.ds(o, PB)], lidx)
        pltpu.async_copy(h.at[idxv], rows, sem).wait()

        def grp(g, _):
            o16 = pl.multiple_of(g * 16, 16)
            dvec = jnp.minimum(lidx[pl.ds(o16, 16)] - srel, RPB)
            for j in range(16):
                d_ = lax.squeeze(lax.slice(dvec, (j,), (j + 1,)), (0,))
                e = o16 + j
                acc[d_, pl.ds(0, 16)] = jnp.maximum(acc[d_, pl.ds(0, 16)],
                                                    rows[e, pl.ds(0, 16)])
                acc[d_, pl.ds(16, 16)] = jnp.maximum(acc[d_, pl.ds(16, 16)],
                                                     rows[e, pl.ds(16, 16)])
            return 0

        lax.fori_loop(0, PB // 16, grp, 0)
        return 0

    lax.fori_loop(0, nblk, blk, 0)
    pltpu.sync_copy(acc.at[pl.ds(0, RPB)],
                    out.at[pl.ds(pl.multiple_of(b * RPB, 8), RPB)])


def _row_spec(d):
    return pl.BlockSpec((TCB, d), lambda i: (i, 0))


def _full_spec(shape):
    return pl.BlockSpec(shape, lambda i: tuple(0 for _ in shape))


def _stage0_body(aggcnt, x, wr, br, wl, h1, cntc):
    a = aggcnt[:, 0:1]
    cnt = aggcnt[:, 1:2]
    h1[...] = jnp.maximum(a * wr[...] + br[...] + x[...] * wl[...], 0.0)
    cntc[...] = jnp.maximum(cnt, 1.0)


def _stage0(aggcnt, x2, wr, br, wl):
    return pl.pallas_call(
        _stage0_body,
        grid=(N // TCB,),
        in_specs=[_row_spec(2), _row_spec(1), _full_spec((1, 32)),
                  _full_spec((1, 32)), _full_spec((1, 32))],
        out_specs=(_row_spec(32), _row_spec(1)),
        out_shape=(jax.ShapeDtypeStruct((N, 32), jnp.float32),
                   jax.ShapeDtypeStruct((N, 1), jnp.float32)),
    )(aggcnt, x2, wr, br, wl)


def _maxpost_body(m, h, wrt, br, wlt, wrnt, hn, yn):
    v = jnp.maximum(jnp.dot(m[...], wrt[...], preferred_element_type=jnp.float32)
                    + br[...]
                    + jnp.dot(h[...], wlt[...], preferred_element_type=jnp.float32),
                    0.0)
    hn[...] = v
    yn[...] = jnp.dot(v, wrnt[...], preferred_element_type=jnp.float32)


def _maxpost(m, h, wrt, br, wlt, wrnt):
    return pl.pallas_call(
        _maxpost_body,
        grid=(N // TCB,),
        in_specs=[_row_spec(32), _row_spec(32), _full_spec((32, 64)),
                  _full_spec((1, 64)), _full_spec((32, 64)),
                  _full_spec((64, 32))],
        out_specs=(_row_spec(64), _row_spec(32)),
        out_shape=(jax.ShapeDtypeStruct((N, 64), jnp.float32),
                   jax.ShapeDtypeStruct((N, 32), jnp.float32)),
    )(m, h, wrt, br, wlt, wrnt)


def _meanpost_body(sres, cntc, h, br, wlt, out):
    out[...] = jnp.maximum(
        sres[...] / cntc[...] + br[...]
        + jnp.dot(h[...], wlt[...], preferred_element_type=jnp.float32), 0.0)


def _meanpost(sres, cntc, h, br, wlt):
    return pl.pallas_call(
        _meanpost_body,
        grid=(N // TCB,),
        in_specs=[_row_spec(32), _row_spec(1), _row_spec(64),
                  _full_spec((1, 32)), _full_spec((64, 32))],
        out_specs=_row_spec(32),
        out_shape=jax.ShapeDtypeStruct((N, 32), jnp.float32),
    )(sres, cntc, h, br, wlt)


def _final_body(m, h, wrt, br, wlt, out):
    out[...] = jax.nn.sigmoid(
        jnp.dot(m[...], wrt[...], preferred_element_type=jnp.float32)
        + br[...]
        + jnp.dot(h[...], wlt[...], preferred_element_type=jnp.float32))


def _final(m, h, wrt, br, wlt):
    return pl.pallas_call(
        _final_body,
        grid=(N // TCB,),
        in_specs=[_row_spec(32), _row_spec(32), _full_spec((32, 1)),
                  _full_spec((1, 1)), _full_spec((32, 1))],
        out_specs=_row_spec(1),
        out_shape=jax.ShapeDtypeStruct((N, 1), jnp.float32),
    )(m, h, wrt, br, wlt)


def kernel(x, edge_index, params):
    p = params
    ei = edge_index.astype(jnp.int32).reshape(-1)
    bsrc, bdst, bcnt = _bucketize(ei)

    z32 = jnp.zeros((ACCR, 32), jnp.float32)
    z16 = jnp.zeros((ACCR, 16), jnp.float32)

    xe = jnp.concatenate([x[:, None], jnp.ones((N, 1), jnp.float32),
                          jnp.zeros((N, 14), jnp.float32)], axis=1)
    aggcnt = _sum16(xe, z16, bsrc, bdst, bcnt)[:N, :2]
    h1, cntc = _stage0(aggcnt, x[:, None],
                       p['Wr0'].reshape(1, 32), p['br0'].reshape(1, 32),
                       p['Wl0'].reshape(1, 32))

    h = h1
    for i in (1, 3, 5):
        m = _max32(h, z32, bsrc, bdst, bcnt)[:N]
        hn, yn = _maxpost(m, h, p['Wr%d' % i].T, p['br%d' % i].reshape(1, 64),
                          p['Wl%d' % i].T, p['Wr%d' % (i + 1)].T)
        s = _sum32(yn, z32, bsrc, bdst, bcnt)[:N]
        h = _meanpost(s, cntc, hn, p['br%d' % (i + 1)].reshape(1, 32),
                      p['Wl%d' % (i + 1)].T)

    m7 = _max32(h, z32, bsrc, bdst, bcnt)[:N]
    return _final(m7, h, p['Wr7'].T, p['br7'].reshape(1, 1), p['Wl7'].T)


# speedup vs baseline: 15.0073x; 1.9066x over previous
import functools

import jax
import jax.numpy as jnp
from jax import lax
from jax.experimental import pallas as pl
from jax.experimental.pallas import tpu as pltpu
from jax.experimental.pallas import tpu_sc as plsc

N = 100000
E = 3200000
NC = 2
NS = 16
NB = NC * NS
RPB = 3128
PADN = NB * RPB
HALFP = NS * RPB
ACCR = 50176
DUMMY = HALFP + 8
MAXR = 3136
BB = 3200
FLUSH = 2048
BUFCAP = 8192
CAP = E + 4096
SUM_PB = {16: 512, 32: 256}
PBM = 384
NGM = PBM // 128
TCB = 1000


def _s0(v):
    return lax.squeeze(lax.slice(v, (0,), (1,)), (0,))


_MESH = plsc.VectorSubcoreMesh(core_axis_name="c", subcore_axis_name="s")


@functools.partial(
    pl.kernel,
    out_type=(
        jax.ShapeDtypeStruct((NB * CAP,), jnp.int32),
        jax.ShapeDtypeStruct((NB * CAP,), jnp.int32),
        jax.ShapeDtypeStruct((NB * 16,), jnp.int32),
    ),
    mesh=_MESH,
    scratch_types=[
        pltpu.VMEM((2, BB), jnp.int32),
        pltpu.VMEM((2, BB), jnp.int32),
        pltpu.VMEM((BUFCAP,), jnp.int32),
        pltpu.VMEM((BUFCAP,), jnp.int32),
        pltpu.VMEM((16,), jnp.int32),
        pltpu.SemaphoreType.DMA,
        pltpu.SemaphoreType.DMA,
    ],
    compiler_params=pltpu.CompilerParams(needs_layout_passes=False),
)
def _bucketize(ei, bsrc, bdst, bcnt, esv, edv, bufs, bufd, cv, semA, semB):
    c = lax.axis_index("c")
    s = lax.axis_index("s")
    b = c * NS + s
    lo = b * RPB
    hi = lo + RPB
    halfbase = c * HALFP
    region = b * CAP
    iota = lax.iota(jnp.int32, 16)
    NBLK = E // BB

    def fetch(blk, p, sem):
        base = pl.multiple_of(blk * BB, 128)
        pltpu.async_copy(ei.at[pl.ds(base, BB)], esv.at[p], sem)
        pltpu.async_copy(ei.at[pl.ds(E + base, BB)], edv.at[p], sem)

    def drain(blk, p, sem):
        base = pl.multiple_of(blk * BB, 128)
        pltpu.make_async_copy(ei.at[pl.ds(base, BB)], esv.at[p], sem).wait()
        pltpu.make_async_copy(ei.at[pl.ds(E + base, BB)], edv.at[p], sem).wait()

    def scan(blk, p, sem, carry):
        off, woff = carry
        drain(blk, p, sem)

        def grp4(i, off_spl):
            for u in range(4):
                o16 = pl.multiple_of((i * 4 + u) * 16, 16)
                sv = esv[p, pl.ds(o16, 16)]
                dv = edv[p, pl.ds(o16, 16)]
                m = (dv >= lo) & (dv < hi)
                mi = m.astype(jnp.int32)
                pos = off_spl + plsc.cumsum(mi) - mi
                plsc.store_scatter(bufs, [pos], sv, mask=m)
                plsc.store_scatter(bufd, [pos], dv - halfbase, mask=m)
                off_spl = off_spl + plsc.all_reduce_population_count(m)
            return off_spl

        off_spl = lax.fori_loop(0, BB // 64, grp4,
                                jnp.full((16,), off, jnp.int32))
        off = _s0(off_spl)

        def flush_body(cw):
            o, w = cw
            dst_off = pl.multiple_of(region + w, 128)
            pltpu.sync_copy(bufs.at[pl.ds(0, FLUSH)], bsrc.at[pl.ds(dst_off, FLUSH)])
            pltpu.sync_copy(bufd.at[pl.ds(0, FLUSH)], bdst.at[pl.ds(dst_off, FLUSH)])

            def shift(i, _):
                o16 = pl.multiple_of(i * 16, 16)
                bufs[pl.ds(o16, 16)] = bufs[pl.ds(FLUSH + o16, 16)]
                bufd[pl.ds(o16, 16)] = bufd[pl.ds(FLUSH + o16, 16)]
                return 0

            lax.fori_loop(0, (BUFCAP - FLUSH) // 16, shift, 0)
            return (o - FLUSH, w + FLUSH)

        return lax.while_loop(lambda cw: cw[0] >= FLUSH, flush_body,
                              (off, woff))

    fetch(0, 0, semA)

    def pair(q, carry):
        k0 = q * 2
        k1 = k0 + 1

        @pl.when(k1 < NBLK)
        def _():
            fetch(k1, 1, semB)

        carry = scan(k0, 0, semA, carry)

        def second(cw):
            @pl.when(k1 + 1 < NBLK)
            def _():
                fetch(k1 + 1, 0, semA)

            return scan(k1, 1, semB, cw)

        return lax.cond(k1 < NBLK, second, lambda cw: cw, carry)

    off, woff = lax.fori_loop(0, (NBLK + 1) // 2, pair,
                              (jnp.int32(0), jnp.int32(0)))
    total = woff + off

    off_spl = jnp.full((16,), off, jnp.int32)
    zeros16 = jnp.zeros((16,), jnp.int32)
    dummy16 = jnp.full((16,), DUMMY, jnp.int32)

    def pad(g, _):
        pos = off_spl + g * 16 + iota
        plsc.store_scatter(bufs, [pos], zeros16)
        plsc.store_scatter(bufd, [pos], dummy16)
        return 0

    lax.fori_loop(0, (2 * FLUSH) // 16, pad, 0)
    dst_off = pl.multiple_of(region + woff, 128)
    pltpu.sync_copy(bufs.at[pl.ds(0, FLUSH)], bsrc.at[pl.ds(dst_off, FLUSH)])
    pltpu.sync_copy(bufd.at[pl.ds(0, FLUSH)], bdst.at[pl.ds(dst_off, FLUSH)])
    dst_off2 = pl.multiple_of(region + woff + FLUSH, 128)
    pltpu.sync_copy(bufs.at[pl.ds(FLUSH, FLUSH)], bsrc.at[pl.ds(dst_off2, FLUSH)])
    pltpu.sync_copy(bufd.at[pl.ds(FLUSH, FLUSH)], bdst.at[pl.ds(dst_off2, FLUSH)])
    cv[...] = jnp.full((16,), total, jnp.int32)
    pltpu.sync_copy(cv, bcnt.at[pl.ds(pl.multiple_of(b * 16, 16), 16)])


def _make_sum(d):
    PBS = SUM_PB[d]
    NGS = PBS // 128
    @functools.partial(
        pl.kernel,
        out_type=jax.ShapeDtypeStruct((PADN, d), jnp.float32),
        mesh=_MESH,
        scratch_types=(
            [pltpu.VMEM_SHARED((ACCR, d), jnp.float32),
             pltpu.VMEM((2, PBS), jnp.int32),
             pltpu.VMEM((2, PBS, d), jnp.float32),
             pltpu.VMEM((16,), jnp.int32)]
            + [pltpu.VMEM((128,), jnp.int32) for _ in range(2 * NGS)]
            + [pltpu.SemaphoreType.DMA] * 4
        ),
        compiler_params=pltpu.CompilerParams(needs_layout_passes=False,
                                             use_tc_tiling_on_sc=False),
    )
    def k(y, zeros, bsrc, bdst, bcnt, out, acc, idxv, rows, cv, *rest):
        dlocs = rest[:2 * NGS]
        semI, semGA, semGB, semS = rest[2 * NGS:]
        semG = (semGA, semGB)
        c = lax.axis_index("c")
        s = lax.axis_index("s")
        b = c * NS + s
        region = b * CAP
        rpt = ACCR // NS
        zo = pl.multiple_of(s * rpt, 8)
        pltpu.sync_copy(zeros.at[pl.ds(zo, rpt)], acc.at[pl.ds(zo, rpt)])
        plsc.subcore_barrier()
        pltpu.sync_copy(bcnt.at[pl.ds(pl.multiple_of(b * 16, 16), 16)], cv)
        cnt = _s0(cv[...])
        nblk = (cnt + PBS - 1) // PBS

        def fetch_fire(kk, p):
            o = pl.multiple_of(region + kk * PBS, 128)
            descs = [pltpu.async_copy(bsrc.at[pl.ds(o, PBS)], idxv.at[p], semI)]
            for j in range(NGS):
                oj = pl.multiple_of(o + j * 128, 128)
                descs.append(pltpu.async_copy(bdst.at[pl.ds(oj, 128)],
                                              dlocs[p * NGS + j], semI))
            for dd in descs:
                dd.wait()
            for j in range(NGS):
                pltpu.async_copy(y.at[idxv.at[p, pl.ds(j * 128, 128)]],
                                 rows.at[p, pl.ds(j * 128, 128)], semG[p])

        def wait_g(p):
            for j in range(NGS):
                pltpu.make_async_copy(
                    y.at[idxv.at[p, pl.ds(j * 128, 128)]],
                    rows.at[p, pl.ds(j * 128, 128)], semG[p]).wait()

        def scatter(p):
            descs = []
            for j in range(NGS):
                descs.append(pltpu.async_copy(
                    rows.at[p, pl.ds(j * 128, 128)],
                    acc.at[dlocs[p * NGS + j]], semS, add=True))
            for dd in descs:
                dd.wait()

        @pl.when(nblk > 0)
        def _():
            fetch_fire(0, 0)

        def pair(q, _):
            k0 = q * 2
            k1 = k0 + 1

            @pl.when(k1 < nblk)
            def _():
                fetch_fire(k1, 1)

            wait_g(0)
            scatter(0)

            @pl.when(k1 < nblk)
            def _():
                @pl.when(k1 + 1 < nblk)
                def _():
                    fetch_fire(k1 + 1, 0)

                wait_g(1)
                scatter(1)

            return 0

        lax.fori_loop(0, (nblk + 1) // 2, pair, 0)
        plsc.subcore_barrier()
        oo = pl.multiple_of(s * RPB, 8)
        pltpu.sync_copy(acc.at[pl.ds(oo, RPB)],
                        out.at[pl.ds(pl.multiple_of(c * HALFP + oo, 8), RPB)])

    return k


_sum16 = _make_sum(16)
_sum32 = _make_sum(32)


@functools.partial(
    pl.kernel,
    out_type=jax.ShapeDtypeStruct((PADN, 32), jnp.float32),
    mesh=_MESH,
    scratch_types=[
        pltpu.VMEM((MAXR, 32), jnp.float32),
        pltpu.VMEM((2, PBM), jnp.int32),
        pltpu.VMEM((2, PBM), jnp.int32),
        pltpu.VMEM((2, PBM, 32), jnp.float32),
        pltpu.VMEM((16,), jnp.int32),
        pltpu.SemaphoreType.DMA,
        pltpu.SemaphoreType.DMA,
        pltpu.SemaphoreType.DMA,
    ],
    compiler_params=pltpu.CompilerParams(needs_layout_passes=False,
                                         use_tc_tiling_on_sc=False),
)
def _max32(h, zeros, bsrc, bdst, bcnt, out, acc, idxv, lidx, rows, cv,
           semI, semGA, semGB):
    c = lax.axis_index("c")
    s = lax.axis_index("s")
    b = c * NS + s
    region = b * CAP
    srel = s * RPB
    semG = (semGA, semGB)
    pltpu.sync_copy(zeros.at[pl.ds(0, MAXR)], acc)
    pltpu.sync_copy(bcnt.at[pl.ds(pl.multiple_of(b * 16, 16), 16)], cv)
    cnt = _s0(cv[...])
    nblk = (cnt + PBM - 1) // PBM

    def fetch_fire(kk, p):
        o = pl.multiple_of(region + kk * PBM, 128)
        d1 = pltpu.async_copy(bsrc.at[pl.ds(o, PBM)], idxv.at[p], semI)
        d2 = pltpu.async_copy(bdst.at[pl.ds(o, PBM)], lidx.at[p], semI)
        d1.wait()
        d2.wait()
        for j in range(NGM):
            pltpu.async_copy(h.at[idxv.at[p, pl.ds(j * 128, 128)]],
                             rows.at[p, pl.ds(j * 128, 128)], semG[p])

    def wait_g(p):
        for j in range(NGM):
            pltpu.make_async_copy(
                h.at[idxv.at[p, pl.ds(j * 128, 128)]],
                rows.at[p, pl.ds(j * 128, 128)], semG[p]).wait()

    def rmw(p):
        def grp(g, _):
            o16 = pl.multiple_of(g * 16, 16)
            dvec = jnp.minimum(lidx[p, pl.ds(o16, 16)] - srel, RPB)
            for j in range(16):
                d_ = lax.squeeze(lax.slice(dvec, (j,), (j + 1,)), (0,))
                e = o16 + j
                acc[d_, pl.ds(0, 16)] = jnp.maximum(acc[d_, pl.ds(0, 16)],
                                                    rows[p, e, pl.ds(0, 16)])
                acc[d_, pl.ds(16, 16)] = jnp.maximum(acc[d_, pl.ds(16, 16)],
                                                     rows[p, e, pl.ds(16, 16)])
            return 0

        lax.fori_loop(0, PBM // 16, grp, 0)

    @pl.when(nblk > 0)
    def _():
        fetch_fire(0, 0)

    def pair(q, _):
        k0 = q * 2
        k1 = k0 + 1

        @pl.when(k1 < nblk)
        def _():
            fetch_fire(k1, 1)

        wait_g(0)
        rmw(0)

        @pl.when(k1 < nblk)
        def _():
            @pl.when(k1 + 1 < nblk)
            def _():
                fetch_fire(k1 + 1, 0)

            wait_g(1)
            rmw(1)

        return 0

    lax.fori_loop(0, (nblk + 1) // 2, pair, 0)
    pltpu.sync_copy(acc.at[pl.ds(0, RPB)],
                    out.at[pl.ds(pl.multiple_of(b * RPB, 8), RPB)])


def _row_spec(d):
    return pl.BlockSpec((TCB, d), lambda i: (i, 0))


def _full_spec(shape):
    return pl.BlockSpec(shape, lambda i: tuple(0 for _ in shape))


def _stage0_body(aggcnt, x, wr, br, wl, h1, cntc):
    a = aggcnt[:, 0:1]
    cnt = aggcnt[:, 1:2]
    h1[...] = jnp.maximum(a * wr[...] + br[...] + x[...] * wl[...], 0.0)
    cntc[...] = jnp.maximum(cnt, 1.0)


def _stage0(aggcnt, x2, wr, br, wl):
    return pl.pallas_call(
        _stage0_body,
        grid=(N // TCB,),
        in_specs=[_row_spec(2), _row_spec(1), _full_spec((1, 32)),
                  _full_spec((1, 32)), _full_spec((1, 32))],
        out_specs=(_row_spec(32), _row_spec(1)),
        out_shape=(jax.ShapeDtypeStruct((N, 32), jnp.float32),
                   jax.ShapeDtypeStruct((N, 1), jnp.float32)),
    )(aggcnt, x2, wr, br, wl)


def _maxpost_body(m, h, wrt, br, wlt, wrnt, hn, yn):
    v = jnp.maximum(jnp.dot(m[...], wrt[...], preferred_element_type=jnp.float32)
                    + br[...]
                    + jnp.dot(h[...], wlt[...], preferred_element_type=jnp.float32),
                    0.0)
    hn[...] = v
    yn[...] = jnp.dot(v, wrnt[...], preferred_element_type=jnp.float32)


def _maxpost(m, h, wrt, br, wlt, wrnt):
    return pl.pallas_call(
        _maxpost_body,
        grid=(N // TCB,),
        in_specs=[_row_spec(32), _row_spec(32), _full_spec((32, 64)),
                  _full_spec((1, 64)), _full_spec((32, 64)),
                  _full_spec((64, 32))],
        out_specs=(_row_spec(64), _row_spec(32)),
        out_shape=(jax.ShapeDtypeStruct((N, 64), jnp.float32),
                   jax.ShapeDtypeStruct((N, 32), jnp.float32)),
    )(m, h, wrt, br, wlt, wrnt)


def _meanpost_body(sres, cntc, h, br, wlt, out):
    out[...] = jnp.maximum(
        sres[...] / cntc[...] + br[...]
        + jnp.dot(h[...], wlt[...], preferred_element_type=jnp.float32), 0.0)


def _meanpost(sres, cntc, h, br, wlt):
    return pl.pallas_call(
        _meanpost_body,
        grid=(N // TCB,),
        in_specs=[_row_spec(32), _row_spec(1), _row_spec(64),
                  _full_spec((1, 32)), _full_spec((64, 32))],
        out_specs=_row_spec(32),
        out_shape=jax.ShapeDtypeStruct((N, 32), jnp.float32),
    )(sres, cntc, h, br, wlt)


def _final_body(m, h, wrt, br, wlt, out):
    out[...] = jax.nn.sigmoid(
        jnp.dot(m[...], wrt[...], preferred_element_type=jnp.float32)
        + br[...]
        + jnp.dot(h[...], wlt[...], preferred_element_type=jnp.float32))


def _final(m, h, wrt, br, wlt):
    return pl.pallas_call(
        _final_body,
        grid=(N // TCB,),
        in_specs=[_row_spec(32), _row_spec(32), _full_spec((32, 1)),
                  _full_spec((1, 1)), _full_spec((32, 1))],
        out_specs=_row_spec(1),
        out_shape=jax.ShapeDtypeStruct((N, 1), jnp.float32),
    )(m, h, wrt, br, wlt)


def kernel(x, edge_index, params):
    p = params
    ei = edge_index.astype(jnp.int32).reshape(-1)
    bsrc, bdst, bcnt = _bucketize(ei)

    z32 = jnp.zeros((ACCR, 32), jnp.float32)
    z16 = jnp.zeros((ACCR, 16), jnp.float32)

    xe = jnp.concatenate([x[:, None], jnp.ones((N, 1), jnp.float32),
                          jnp.zeros((N, 14), jnp.float32)], axis=1)
    aggcnt = _sum16(xe, z16, bsrc, bdst, bcnt)[:N, :2]
    h1, cntc = _stage0(aggcnt, x[:, None],
                       p['Wr0'].reshape(1, 32), p['br0'].reshape(1, 32),
                       p['Wl0'].reshape(1, 32))

    h = h1
    for i in (1, 3, 5):
        m = _max32(h, z32, bsrc, bdst, bcnt)[:N]
        hn, yn = _maxpost(m, h, p['Wr%d' % i].T, p['br%d' % i].reshape(1, 64),
                          p['Wl%d' % i].T, p['Wr%d' % (i + 1)].T)
        s = _sum32(yn, z32, bsrc, bdst, bcnt)[:N]
        h = _meanpost(s, cntc, hn, p['br%d' % (i + 1)].reshape(1, 32),
                      p['Wl%d' % (i + 1)].T)

    m7 = _max32(h, z32, bsrc, bdst, bcnt)[:N]
    return _final(m7, h, p['Wr7'].T, p['br7'].reshape(1, 1), p['Wl7'].T)
